# baseline (speedup 1.0000x reference)
import functools

import jax
import jax.numpy as jnp
from jax import lax
from jax.experimental import pallas as pl
from jax.experimental.pallas import tpu as pltpu
from jax.experimental.pallas import tpu_sc as plsc

N_EMB = 1_000_000
D_EMB = 64
BATCH = 16384

_NC = 2
_NS = 16
_NW = _NC * _NS
_CH = 128
_NCH = BATCH // (_NW * _CH)
_NCHUNKS = BATCH // _CH

_mesh = plsc.VectorSubcoreMesh(core_axis_name="c", subcore_axis_name="s")


@functools.partial(
    pl.kernel,
    mesh=_mesh,
    compiler_params=pltpu.CompilerParams(use_tc_tiling_on_sc=False),
    out_type=jax.ShapeDtypeStruct((_NCHUNKS, _CH, D_EMB), jnp.float32),
    scratch_types=[
        pltpu.VMEM((_NCH, _CH), jnp.int32),
        pltpu.VMEM((_CH, D_EMB), jnp.float32),
        pltpu.VMEM((_CH, D_EMB), jnp.float32),
        pltpu.VMEM((_CH, D_EMB), jnp.float32),
        pltpu.VMEM((_CH, D_EMB), jnp.float32),
        pltpu.SemaphoreType.DMA,
        pltpu.SemaphoreType.DMA,
        pltpu.SemaphoreType.DMA,
        pltpu.SemaphoreType.DMA,
        pltpu.SemaphoreType.DMA,
    ],
)
def _emb_lookup(idx_hbm, table_hbm, out_hbm,
                idx_v, b0, b1, b2, b3, sg0, sg1, sg2, sg3, so):
    wid = lax.axis_index("s") * _NC + lax.axis_index("c")
    base = wid * _NCH
    pltpu.sync_copy(idx_hbm.at[pl.ds(base, _NCH)], idx_v)
    bufs = (b0, b1, b2, b3)
    sems = (sg0, sg1, sg2, sg3)
    gathers = [
        pltpu.async_copy(table_hbm.at[idx_v.at[j]], bufs[j], sems[j])
        for j in range(_NCH)
    ]
    writes = []
    for j in range(_NCH):
        gathers[j].wait()
        writes.append(pltpu.async_copy(bufs[j], out_hbm.at[base + j], so))
    for w in writes:
        w.wait()


def kernel(x, emb):
    idx = x.astype(jnp.int32).reshape(_NCHUNKS, _CH)
    out = _emb_lookup(idx, emb)
    return out.reshape(BATCH, D_EMB)
